# Initial kernel scaffold; baseline (speedup 1.0000x reference)
import jax
import jax.numpy as jnp
from jax.experimental import pallas as pl


def kernel(input_enc, emb_table, ln_gamma, ln_beta, embedding_index):
    raise NotImplementedError("write your pallas kernel here")


# speedup vs baseline: 3.4017x; 3.4017x over previous
import functools

import jax
import jax.numpy as jnp
from jax.experimental import pallas as pl
from jax.experimental.pallas import tpu as pltpu

L = 32768
H = 1024
K = 4
EPS = 1e-12
BLOCK = 512


def _ln_body(idx_ref, in_ref, tab_ref, gam_ref, bet_ref, out_ref):
    idx = idx_ref[0, 0, :]
    x = in_ref[...]
    tab = tab_ref[...]
    ks = jax.lax.broadcasted_iota(jnp.int32, (BLOCK, K), 1)
    onehot = (idx[:, None] == ks).astype(jnp.float32)
    pos = jnp.dot(onehot, tab, preferred_element_type=jnp.float32)
    x = x + pos
    mu = jnp.mean(x, axis=-1, keepdims=True)
    xc = x - mu
    var = jnp.mean(xc * xc, axis=-1, keepdims=True)
    y = xc * jax.lax.rsqrt(var + EPS)
    out_ref[...] = y * gam_ref[...] + bet_ref[...]


@jax.jit
def kernel(input_enc, emb_table, ln_gamma, ln_beta, embedding_index):
    nb = L // BLOCK
    idx3 = embedding_index.astype(jnp.int32).reshape(nb, 1, BLOCK)
    gam = ln_gamma.reshape(1, H)
    bet = ln_beta.reshape(1, H)
    return pl.pallas_call(
        _ln_body,
        grid=(nb,),
        in_specs=[
            pl.BlockSpec((1, 1, BLOCK), lambda i: (i, 0, 0)),
            pl.BlockSpec((BLOCK, H), lambda i: (i, 0)),
            pl.BlockSpec((K, H), lambda i: (0, 0)),
            pl.BlockSpec((1, H), lambda i: (0, 0)),
            pl.BlockSpec((1, H), lambda i: (0, 0)),
        ],
        out_specs=pl.BlockSpec((BLOCK, H), lambda i: (i, 0)),
        out_shape=jax.ShapeDtypeStruct((L, H), jnp.float32),
    )(idx3, input_enc, emb_table, gam, bet)


# speedup vs baseline: 4.1051x; 1.2068x over previous
import functools

import jax
import jax.numpy as jnp
from jax.experimental import pallas as pl
from jax.experimental.pallas import tpu as pltpu

L = 32768
H = 1024
K = 4
EPS = 1e-12
BLOCK = 1024


def _ln_body(idx_ref, in_ref, tab_ref, gam_ref, bet_ref, out_ref):
    idx = idx_ref[0, 0, :]
    x = in_ref[...]
    tab = tab_ref[...]
    ks = jax.lax.broadcasted_iota(jnp.int32, (BLOCK, K), 1)
    onehot = (idx[:, None] == ks).astype(jnp.float32)
    pos = jnp.dot(onehot, tab, preferred_element_type=jnp.float32)
    x = x + pos
    mu = jnp.mean(x, axis=-1, keepdims=True)
    xc = x - mu
    var = jnp.mean(xc * xc, axis=-1, keepdims=True)
    y = xc * jax.lax.rsqrt(var + EPS)
    out_ref[...] = y * gam_ref[...] + bet_ref[...]


@jax.jit
def kernel(input_enc, emb_table, ln_gamma, ln_beta, embedding_index):
    nb = L // BLOCK
    idx3 = embedding_index.astype(jnp.int32).reshape(nb, 1, BLOCK)
    gam = ln_gamma.reshape(1, H)
    bet = ln_beta.reshape(1, H)
    return pl.pallas_call(
        _ln_body,
        grid=(nb,),
        in_specs=[
            pl.BlockSpec((1, 1, BLOCK), lambda i: (i, 0, 0)),
            pl.BlockSpec((BLOCK, H), lambda i: (i, 0)),
            pl.BlockSpec((K, H), lambda i: (0, 0)),
            pl.BlockSpec((1, H), lambda i: (0, 0)),
            pl.BlockSpec((1, H), lambda i: (0, 0)),
        ],
        out_specs=pl.BlockSpec((BLOCK, H), lambda i: (i, 0)),
        out_shape=jax.ShapeDtypeStruct((L, H), jnp.float32),
    )(idx3, input_enc, emb_table, gam, bet)


# speedup vs baseline: 4.3926x; 1.0701x over previous
import functools

import jax
import jax.numpy as jnp
from jax.experimental import pallas as pl
from jax.experimental.pallas import tpu as pltpu

L = 32768
H = 1024
K = 4
EPS = 1e-12
BLOCK = 2048


def _ln_body(idx_ref, in_ref, tab_ref, gam_ref, bet_ref, out_ref):
    idx = idx_ref[0, 0, :]
    x = in_ref[...]
    tab = tab_ref[...]
    ks = jax.lax.broadcasted_iota(jnp.int32, (BLOCK, K), 1)
    onehot = (idx[:, None] == ks).astype(jnp.float32)
    pos = jnp.dot(onehot, tab, preferred_element_type=jnp.float32)
    x = x + pos
    mu = jnp.mean(x, axis=-1, keepdims=True)
    xc = x - mu
    var = jnp.mean(xc * xc, axis=-1, keepdims=True)
    y = xc * jax.lax.rsqrt(var + EPS)
    out_ref[...] = y * gam_ref[...] + bet_ref[...]


@jax.jit
def kernel(input_enc, emb_table, ln_gamma, ln_beta, embedding_index):
    nb = L // BLOCK
    idx3 = embedding_index.astype(jnp.int32).reshape(nb, 1, BLOCK)
    gam = ln_gamma.reshape(1, H)
    bet = ln_beta.reshape(1, H)
    return pl.pallas_call(
        _ln_body,
        grid=(nb,),
        in_specs=[
            pl.BlockSpec((1, 1, BLOCK), lambda i: (i, 0, 0)),
            pl.BlockSpec((BLOCK, H), lambda i: (i, 0)),
            pl.BlockSpec((K, H), lambda i: (0, 0)),
            pl.BlockSpec((1, H), lambda i: (0, 0)),
            pl.BlockSpec((1, H), lambda i: (0, 0)),
        ],
        out_specs=pl.BlockSpec((BLOCK, H), lambda i: (i, 0)),
        out_shape=jax.ShapeDtypeStruct((L, H), jnp.float32),
    )(idx3, input_enc, emb_table, gam, bet)
